# Initial kernel scaffold; baseline (speedup 1.0000x reference)
import jax
import jax.numpy as jnp
from jax.experimental import pallas as pl


def kernel(x, w_e0, b_e0, w_e1, b_e1, w_d0, b_d0, w_d1, b_d1):
    raise NotImplementedError("write your pallas kernel here")


# speedup vs baseline: 1.2226x; 1.2226x over previous
import functools

import jax
import jax.numpy as jnp
from jax import lax
from jax.experimental import pallas as pl
from jax.experimental.pallas import tpu as pltpu


def _build_body(num_enc, num_dec, T, future_len, H, W, CP, S, NQ):
    L = num_enc + num_dec
    PW = W + 2
    NR = H * PW
    C2 = 2 * CP
    G4 = 4 * CP
    K9 = 9 * C2
    deltas = [0, -PW - 1, -PW, -PW + 1, -1, 1, PW - 1, PW, PW + 1]

    def body(x_ref, w_ref, b_ref, o_ref, h_ref, c_ref, q_ref):
        h_ref[...] = jnp.zeros_like(h_ref)
        c_ref[...] = jnp.zeros_like(c_ref)
        q_ref[...] = jnp.zeros_like(q_ref)

        lane = lax.broadcasted_iota(jnp.int32, (1, G4), 1)
        sig = lane < 3 * CP
        sc = jnp.where(sig, 0.5, 1.0)
        sb = jnp.where(sig, 0.5, 0.0)
        rr = lax.broadcasted_iota(jnp.int32, (NR, 1), 0) % PW
        col_ok = jnp.logical_and(rr != 0, rr != PW - 1)

        def cell(layer, inp_bf16):
            q_ref[S:S + NR, 0:CP] = inp_bf16
            q_ref[S:S + NR, CP:C2] = h_ref[layer].astype(jnp.bfloat16)
            for gi in range(1, 9):
                d = deltas[gi]
                q_ref[S:S + NR, C2 * gi:C2 * (gi + 1)] = (
                    q_ref[S + d:S + d + NR, 0:C2])
            gates = jnp.dot(q_ref[S:S + NR, :], w_ref[layer],
                            preferred_element_type=jnp.float32) + b_ref[layer]
            act = jnp.tanh(gates * sc) * sc + sb
            i_g = act[:, 0 * CP:1 * CP]
            f_g = act[:, 1 * CP:2 * CP]
            o_g = act[:, 2 * CP:3 * CP]
            g_g = act[:, 3 * CP:4 * CP]
            c_n = f_g * c_ref[layer] + i_g * g_g
            h_ref[layer] = jnp.where(col_ok, o_g * jnp.tanh(c_n), 0.0)
            c_ref[layer] = c_n

        def enc_step(t, carry):
            frame = x_ref[0, t].reshape(NR, CP)
            cell(0, frame)
            for i in range(1, num_enc):
                cell(i, h_ref[i - 1].astype(jnp.bfloat16))
            return carry

        lax.fori_loop(0, T, enc_step, 0)

        def dec_step(j, carry):
            cell(num_enc, h_ref[num_enc - 1].astype(jnp.bfloat16))
            for d in range(1, num_dec):
                layer = num_enc + d
                cell(layer, h_ref[layer - 1].astype(jnp.bfloat16))
            o_ref[0, j] = h_ref[L - 1].reshape(H, PW, CP)
            return carry

        lax.fori_loop(0, future_len, dec_step, 0)

    return body


def _pack(layers, CP):
    PW_taps = [(1, 1), (0, 0), (0, 1), (0, 2), (1, 0), (1, 2),
               (2, 0), (2, 1), (2, 2)]
    L = len(layers)
    C2 = 2 * CP
    w_all = jnp.zeros((L, 9 * C2, 4 * CP), jnp.float32)
    b_all = jnp.zeros((L, 1, 4 * CP), jnp.float32)
    for l, (w, b) in enumerate(layers):
        ctot, ch = w.shape[-2], w.shape[-1]
        cin = ctot - ch
        for gi, (ky, kx) in enumerate(PW_taps):
            wt = jnp.transpose(w[:, ky, kx], (1, 0, 2))
            wt = jnp.pad(wt, ((0, 0), (0, 0), (0, CP - ch)))
            wt = wt.reshape(ctot, 4 * CP)
            r0 = C2 * gi
            w_all = w_all.at[l, r0:r0 + cin, :].set(wt[:cin])
            w_all = w_all.at[l, r0 + CP:r0 + CP + ch, :].set(wt[cin:])
        bb = jnp.pad(b.reshape(4, ch), ((0, 0), (0, CP - ch))).reshape(4 * CP)
        b_all = b_all.at[l, 0].set(bb)
    return w_all.astype(jnp.bfloat16), b_all


def _forward(x, enc_params, dec_params, future_len):
    B, c_in, T, H, W = x.shape
    num_enc, num_dec = len(enc_params), len(dec_params)
    L = num_enc + num_dec
    hidden = enc_params[0][0].shape[-1]
    c_out = dec_params[-1][0].shape[-1]
    CP = max(c_in, hidden, c_out)
    CP = ((CP + 31) // 32) * 32
    PW = W + 2
    NR = H * PW
    K9 = 9 * 2 * CP
    IMG0 = (-PW) % 16
    S = IMG0 + PW
    NQ = ((S + NR + PW + 1) + 15) // 16 * 16

    w_all, b_all = _pack(list(enc_params) + list(dec_params), CP)

    x_l = jnp.transpose(x, (0, 2, 3, 4, 1))
    x_l = jnp.pad(x_l, ((0, 0), (0, 0), (0, 0), (1, 1), (0, CP - c_in)))
    x_l = x_l.astype(jnp.bfloat16)

    body = _build_body(num_enc, num_dec, T, future_len, H, W, CP, S, NQ)

    out = pl.pallas_call(
        body,
        out_shape=jax.ShapeDtypeStruct((B, future_len, H, PW, CP),
                                       jnp.float32),
        grid=(B,),
        in_specs=[
            pl.BlockSpec((1, T, H, PW, CP), lambda b: (b, 0, 0, 0, 0)),
            pl.BlockSpec((L, K9, 4 * CP), lambda b: (0, 0, 0)),
            pl.BlockSpec((L, 1, 4 * CP), lambda b: (0, 0, 0)),
        ],
        out_specs=pl.BlockSpec((1, future_len, H, PW, CP),
                               lambda b: (b, 0, 0, 0, 0)),
        scratch_shapes=[
            pltpu.VMEM((L, NR, CP), jnp.float32),
            pltpu.VMEM((L, NR, CP), jnp.float32),
            pltpu.VMEM((NQ, K9), jnp.bfloat16),
        ],
        compiler_params=pltpu.CompilerParams(
            dimension_semantics=("parallel",),
            vmem_limit_bytes=64 * 1024 * 1024),
    )(x_l, w_all, b_all)

    return [jnp.transpose(out[:, j, :, 1:W + 1, :c_out], (0, 3, 1, 2))
            for j in range(future_len)]


def kernel(x, w_e0, b_e0, w_e1, b_e1, w_d0, b_d0, w_d1, b_d1):
    enc_params = [(w_e0, b_e0), (w_e1, b_e1)]
    dec_params = [(w_d0, b_d0), (w_d1, b_d1)]
    return _forward(x, enc_params, dec_params, future_len=10)
